# Initial kernel scaffold; baseline (speedup 1.0000x reference)
import jax
import jax.numpy as jnp
from jax.experimental import pallas as pl


def kernel(x, fov_mask, scene_embed, W_proj, b_proj, ln_g, ln_b, W1, b1, W2, b2):
    raise NotImplementedError("write your pallas kernel here")


# speedup vs baseline: 1.0521x; 1.0521x over previous
import functools

import jax
import jax.numpy as jnp
from jax.experimental import pallas as pl
from jax.experimental.pallas import tpu as pltpu


def _i2st_block(x_ref, m_ref, se_ref, wp_ref, bp_ref, g_ref, lb_ref,
                w1_ref, b1_ref, w2_ref, b2_ref, out_ref):
    x = x_ref[...]
    proj = jnp.dot(x, wp_ref[...], preferred_element_type=jnp.float32)
    proj = proj + bp_ref[...]
    m = m_ref[...]
    scene = jnp.where(m > 0, proj, se_ref[...])
    mu = jnp.mean(scene, axis=-1, keepdims=True)
    cen = scene - mu
    var = jnp.mean(cen * cen, axis=-1, keepdims=True)
    h = cen * jax.lax.rsqrt(var + 1e-5) * g_ref[...] + lb_ref[...]
    ff = jnp.dot(h, w1_ref[...], preferred_element_type=jnp.float32)
    ff = jax.nn.gelu(ff + b1_ref[...])
    ff = jnp.dot(ff, w2_ref[...], preferred_element_type=jnp.float32)
    out_ref[...] = h + ff + b2_ref[...]


@functools.partial(jax.jit, static_argnames=("block_n",))
def _i2st(x, mask, scene_embed, W_proj, b_proj, ln_g, ln_b, W1, b1, W2, b2,
          block_n=1024):
    n, c = x.shape
    h_dim = W1.shape[1]
    grid = (n // block_n,)
    row_spec = pl.BlockSpec((block_n, c), lambda i: (i, 0))
    full = lambda a: pl.BlockSpec(a.shape, lambda i: (0,) * a.ndim)
    return pl.pallas_call(
        _i2st_block,
        grid=grid,
        in_specs=[
            row_spec,
            pl.BlockSpec((block_n, 1), lambda i: (i, 0)),
            row_spec,
            full(W_proj), full(b_proj), full(ln_g), full(ln_b),
            full(W1), full(b1), full(W2), full(b2),
        ],
        out_specs=row_spec,
        out_shape=jax.ShapeDtypeStruct((n, c), jnp.float32),
    )(x, mask, scene_embed, W_proj, b_proj, ln_g, ln_b, W1, b1, W2, b2)


def kernel(x, fov_mask, scene_embed, W_proj, b_proj, ln_g, ln_b, W1, b1, W2, b2):
    b, n, c = x.shape
    h_dim = W1.shape[1]
    x2 = x.reshape(b * n, c)
    mask = fov_mask.reshape(b * n, 1).astype(jnp.float32)
    se = jnp.broadcast_to(scene_embed[None], (b, n, c)).reshape(b * n, c)
    out = _i2st(x2, mask, se,
                W_proj, b_proj.reshape(1, c), ln_g.reshape(1, c),
                ln_b.reshape(1, c), W1, b1.reshape(1, h_dim), W2,
                b2.reshape(1, c))
    return out.reshape(b, n, c)


# speedup vs baseline: 1.1609x; 1.1035x over previous
import functools

import jax
import jax.numpy as jnp
from jax.experimental import pallas as pl
from jax.experimental.pallas import tpu as pltpu


def _i2st_block(x_ref, m_ref, se_ref, wp_ref, bp_ref, g_ref, lb_ref,
                w1_ref, b1_ref, w2_ref, b2_ref, out_ref):
    bf = jnp.bfloat16
    x = x_ref[...]
    proj = jnp.dot(x.astype(bf), wp_ref[...].astype(bf),
                   preferred_element_type=jnp.float32)
    proj = proj + bp_ref[...]
    m = m_ref[...]
    scene = jnp.where(m > 0, proj, se_ref[...])
    mu = jnp.mean(scene, axis=-1, keepdims=True)
    cen = scene - mu
    var = jnp.mean(cen * cen, axis=-1, keepdims=True)
    h = cen * jax.lax.rsqrt(var + 1e-5) * g_ref[...] + lb_ref[...]
    ff = jnp.dot(h.astype(bf), w1_ref[...].astype(bf),
                 preferred_element_type=jnp.float32)
    ff = jax.nn.gelu(ff + b1_ref[...])
    ff = jnp.dot(ff.astype(bf), w2_ref[...].astype(bf),
                 preferred_element_type=jnp.float32)
    out_ref[...] = h + ff + b2_ref[...]


@functools.partial(jax.jit, static_argnames=("block_n",))
def _i2st(x, mask, scene_embed, W_proj, b_proj, ln_g, ln_b, W1, b1, W2, b2,
          block_n=1024):
    n, c = x.shape
    h_dim = W1.shape[1]
    grid = (n // block_n,)
    row_spec = pl.BlockSpec((block_n, c), lambda i: (i, 0))
    full = lambda a: pl.BlockSpec(a.shape, lambda i: (0,) * a.ndim)
    return pl.pallas_call(
        _i2st_block,
        grid=grid,
        in_specs=[
            row_spec,
            pl.BlockSpec((block_n, 1), lambda i: (i, 0)),
            row_spec,
            full(W_proj), full(b_proj), full(ln_g), full(ln_b),
            full(W1), full(b1), full(W2), full(b2),
        ],
        out_specs=row_spec,
        out_shape=jax.ShapeDtypeStruct((n, c), jnp.float32),
    )(x, mask, scene_embed, W_proj, b_proj, ln_g, ln_b, W1, b1, W2, b2)


def kernel(x, fov_mask, scene_embed, W_proj, b_proj, ln_g, ln_b, W1, b1, W2, b2):
    b, n, c = x.shape
    h_dim = W1.shape[1]
    x2 = x.reshape(b * n, c)
    mask = fov_mask.reshape(b * n, 1).astype(jnp.float32)
    se = jnp.broadcast_to(scene_embed[None], (b, n, c)).reshape(b * n, c)
    out = _i2st(x2, mask, se,
                W_proj, b_proj.reshape(1, c), ln_g.reshape(1, c),
                ln_b.reshape(1, c), W1, b1.reshape(1, h_dim), W2,
                b2.reshape(1, c))
    return out.reshape(b, n, c)


# speedup vs baseline: 1.2921x; 1.1130x over previous
import functools

import jax
import jax.numpy as jnp
from jax.experimental import pallas as pl
from jax.experimental.pallas import tpu as pltpu


def _i2st_block(x_ref, m_ref, se_ref, wp_ref, bp_ref, g_ref, lb_ref,
                w1_ref, b1_ref, w2_ref, b2_ref, out_ref):
    bf = jnp.bfloat16
    x = x_ref[...]
    proj = jnp.dot(x.astype(bf), wp_ref[...].astype(bf),
                   preferred_element_type=jnp.float32)
    proj = proj + bp_ref[...]
    m = m_ref[...]
    scene = jnp.where(m > 0, proj, se_ref[...])
    mu = jnp.mean(scene, axis=-1, keepdims=True)
    cen = scene - mu
    var = jnp.mean(cen * cen, axis=-1, keepdims=True)
    h = cen * jax.lax.rsqrt(var + 1e-5) * g_ref[...] + lb_ref[...]
    ff = jnp.dot(h.astype(bf), w1_ref[...].astype(bf),
                 preferred_element_type=jnp.float32)
    ff = ff + b1_ref[...]
    a = 0.7978845608028654
    b = a * 0.044715
    half = 0.5 * ff
    ff = half + half * jnp.tanh(ff * (b * ff * ff + a))
    ff = jnp.dot(ff.astype(bf), w2_ref[...].astype(bf),
                 preferred_element_type=jnp.float32)
    out_ref[...] = h + ff + b2_ref[...]


@functools.partial(jax.jit, static_argnames=("block_n",))
def _i2st(x, mask, scene_embed, W_proj, b_proj, ln_g, ln_b, W1, b1, W2, b2,
          block_n=2048):
    n, c = x.shape
    h_dim = W1.shape[1]
    grid = (n // block_n,)
    row_spec = pl.BlockSpec((block_n, c), lambda i: (i, 0))
    full = lambda a: pl.BlockSpec(a.shape, lambda i: (0,) * a.ndim)
    return pl.pallas_call(
        _i2st_block,
        grid=grid,
        in_specs=[
            row_spec,
            pl.BlockSpec((block_n, 1), lambda i: (i, 0)),
            row_spec,
            full(W_proj), full(b_proj), full(ln_g), full(ln_b),
            full(W1), full(b1), full(W2), full(b2),
        ],
        out_specs=row_spec,
        out_shape=jax.ShapeDtypeStruct((n, c), jnp.float32),
    )(x, mask, scene_embed, W_proj, b_proj, ln_g, ln_b, W1, b1, W2, b2)


def kernel(x, fov_mask, scene_embed, W_proj, b_proj, ln_g, ln_b, W1, b1, W2, b2):
    b, n, c = x.shape
    h_dim = W1.shape[1]
    x2 = x.reshape(b * n, c)
    mask = fov_mask.reshape(b * n, 1).astype(jnp.float32)
    se = jnp.broadcast_to(scene_embed[None], (b, n, c)).reshape(b * n, c)
    out = _i2st(x2, mask, se,
                W_proj, b_proj.reshape(1, c), ln_g.reshape(1, c),
                ln_b.reshape(1, c), W1, b1.reshape(1, h_dim), W2,
                b2.reshape(1, c))
    return out.reshape(b, n, c)


# speedup vs baseline: 1.3987x; 1.0825x over previous
import functools

import jax
import jax.numpy as jnp
from jax.experimental import pallas as pl
from jax.experimental.pallas import tpu as pltpu


def _i2st_block(x_ref, m_ref, se_ref, wp_ref, bp_ref, g_ref, lb_ref,
                w1_ref, b1_ref, w2_ref, b2_ref, out_ref):
    bf = jnp.bfloat16
    proj = jnp.dot(x_ref[...], wp_ref[...].astype(bf),
                   preferred_element_type=jnp.float32)
    proj = proj + bp_ref[...]
    m = m_ref[...]
    scene = jnp.where(m > 0, proj, se_ref[...])
    mu = jnp.mean(scene, axis=-1, keepdims=True)
    cen = scene - mu
    var = jnp.mean(cen * cen, axis=-1, keepdims=True)
    h = cen * jax.lax.rsqrt(var + 1e-5) * g_ref[...] + lb_ref[...]
    a = jnp.asarray(0.7978845608028654, bf)
    b = jnp.asarray(0.7978845608028654 * 0.044715, bf)
    hb = h.astype(bf)
    w1 = w1_ref[...].astype(bf)
    w2 = w2_ref[...].astype(bf)
    b1 = b1_ref[...].astype(bf)
    acc = h + b2_ref[...]
    n_chunks = 4
    ck = w1.shape[1] // n_chunks
    for k in range(n_chunks):
        ffk = jnp.dot(hb, w1[:, k * ck:(k + 1) * ck],
                      preferred_element_type=jnp.float32)
        ffk = ffk.astype(bf) + b1[:, k * ck:(k + 1) * ck]
        half = jnp.asarray(0.5, bf) * ffk
        gk = half + half * jnp.tanh(ffk * (b * ffk * ffk + a))
        acc = acc + jnp.dot(gk, w2[k * ck:(k + 1) * ck, :],
                            preferred_element_type=jnp.float32)
    out_ref[...] = acc


@functools.partial(jax.jit, static_argnames=("block_n",))
def _i2st(x, mask, scene_embed, W_proj, b_proj, ln_g, ln_b, W1, b1, W2, b2,
          block_n=2048):
    n, c = x.shape
    h_dim = W1.shape[1]
    grid = (n // block_n,)
    row_spec = pl.BlockSpec((block_n, c), lambda i: (i, 0))
    full = lambda a: pl.BlockSpec(a.shape, lambda i: (0,) * a.ndim)
    return pl.pallas_call(
        _i2st_block,
        grid=grid,
        in_specs=[
            row_spec,
            pl.BlockSpec((block_n, 1), lambda i: (i, 0)),
            row_spec,
            full(W_proj), full(b_proj), full(ln_g), full(ln_b),
            full(W1), full(b1), full(W2), full(b2),
        ],
        out_specs=row_spec,
        out_shape=jax.ShapeDtypeStruct((n, c), jnp.float32),
    )(x, mask, scene_embed, W_proj, b_proj, ln_g, ln_b, W1, b1, W2, b2)


def kernel(x, fov_mask, scene_embed, W_proj, b_proj, ln_g, ln_b, W1, b1, W2, b2):
    b, n, c = x.shape
    h_dim = W1.shape[1]
    x2 = x.reshape(b * n, c).astype(jnp.bfloat16)
    mask = fov_mask.reshape(b * n, 1).astype(jnp.float32)
    se = jnp.broadcast_to(scene_embed[None], (b, n, c)).reshape(b * n, c)
    out = _i2st(x2, mask, se,
                W_proj, b_proj.reshape(1, c), ln_g.reshape(1, c),
                ln_b.reshape(1, c), W1, b1.reshape(1, h_dim), W2,
                b2.reshape(1, c))
    return out.reshape(b, n, c)


# speedup vs baseline: 1.4812x; 1.0590x over previous
import functools

import jax
import jax.numpy as jnp
from jax.experimental import pallas as pl
from jax.experimental.pallas import tpu as pltpu


_ROW_SPLIT = 1
_H_CHUNKS = 4


def _i2st_block(x_ref, m_ref, se_ref, wp_ref, bp_ref, g_ref, lb_ref,
                w1_ref, b1_ref, w2_ref, b2_ref, out_ref):
    bf = jnp.bfloat16
    wp = wp_ref[...]
    w1 = w1_ref[...]
    w2 = w2_ref[...]
    b1 = b1_ref[...]
    a = jnp.asarray(0.7978845608028654, bf)
    b = jnp.asarray(0.7978845608028654 * 0.044715, bf)
    rows = x_ref.shape[0] // _ROW_SPLIT
    ck = w1.shape[1] // _H_CHUNKS
    for r in range(_ROW_SPLIT):
        sl = pl.ds(r * rows, rows)
        proj = jnp.dot(x_ref[sl, :], wp, preferred_element_type=jnp.float32)
        proj = proj + bp_ref[...]
        scene = jnp.where(m_ref[sl, :] > 0, proj, se_ref[sl, :])
        mu = jnp.mean(scene, axis=-1, keepdims=True)
        cen = scene - mu
        var = jnp.mean(cen * cen, axis=-1, keepdims=True)
        h = cen * jax.lax.rsqrt(var + 1e-5) * g_ref[...] + lb_ref[...]
        hb = h.astype(bf)
        acc = h + b2_ref[...]
        for k in range(_H_CHUNKS):
            ffk = jnp.dot(hb, w1[:, k * ck:(k + 1) * ck],
                          preferred_element_type=jnp.float32)
            ffk = ffk.astype(bf) + b1[:, k * ck:(k + 1) * ck]
            half = jnp.asarray(0.5, bf) * ffk
            gk = half + half * jnp.tanh(ffk * (b * ffk * ffk + a))
            acc = acc + jnp.dot(gk, w2[k * ck:(k + 1) * ck, :],
                                preferred_element_type=jnp.float32)
        out_ref[sl, :] = acc


@functools.partial(jax.jit, static_argnames=("block_n",))
def _i2st(x, mask, scene_embed, W_proj, b_proj, ln_g, ln_b, W1, b1, W2, b2,
          block_n=4096):
    n, c = x.shape
    h_dim = W1.shape[1]
    grid = (n // block_n,)
    row_spec = pl.BlockSpec((block_n, c), lambda i: (i, 0))
    full = lambda a: pl.BlockSpec(a.shape, lambda i: (0,) * a.ndim)
    return pl.pallas_call(
        _i2st_block,
        grid=grid,
        in_specs=[
            row_spec,
            pl.BlockSpec((block_n, 1), lambda i: (i, 0)),
            row_spec,
            full(W_proj), full(b_proj), full(ln_g), full(ln_b),
            full(W1), full(b1), full(W2), full(b2),
        ],
        out_specs=row_spec,
        out_shape=jax.ShapeDtypeStruct((n, c), jnp.float32),
    )(x, mask, scene_embed, W_proj, b_proj, ln_g, ln_b, W1, b1, W2, b2)


def kernel(x, fov_mask, scene_embed, W_proj, b_proj, ln_g, ln_b, W1, b1, W2, b2):
    b, n, c = x.shape
    h_dim = W1.shape[1]
    x2 = x.reshape(b * n, c).astype(jnp.bfloat16)
    mask = fov_mask.reshape(b * n, 1).astype(jnp.float32)
    se = jnp.broadcast_to(scene_embed[None], (b, n, c)).reshape(b * n, c)
    bf = jnp.bfloat16
    out = _i2st(x2, mask, se,
                W_proj.astype(bf), b_proj.reshape(1, c), ln_g.reshape(1, c),
                ln_b.reshape(1, c), W1.astype(bf), b1.reshape(1, h_dim).astype(bf),
                W2.astype(bf), b2.reshape(1, c))
    return out.reshape(b, n, c)


# speedup vs baseline: 1.4884x; 1.0049x over previous
import functools

import jax
import jax.numpy as jnp
from jax.experimental import pallas as pl
from jax.experimental.pallas import tpu as pltpu


_ROW_SPLIT = 1
_H_CHUNKS = 4


def _i2st_block(x_ref, m_ref, se_ref, wp_ref, bp_ref, g_ref, lb_ref,
                w1_ref, b1_ref, w2_ref, b2_ref, out_ref):
    bf = jnp.bfloat16
    wp = wp_ref[...]
    w1 = w1_ref[...]
    w2 = w2_ref[...]
    b1 = b1_ref[...]
    a = jnp.asarray(0.7978845608028654, bf)
    b = jnp.asarray(0.7978845608028654 * 0.044715, bf)
    rows = x_ref.shape[0] // _ROW_SPLIT
    ck = w1.shape[1] // _H_CHUNKS
    for r in range(_ROW_SPLIT):
        sl = pl.ds(r * rows, rows)
        proj = jnp.dot(x_ref[sl, :], wp, preferred_element_type=jnp.float32)
        proj = proj + bp_ref[...]
        scene = jnp.where(m_ref[sl, :] > 0, proj, se_ref[sl, :])
        mu = jnp.mean(scene, axis=-1, keepdims=True)
        cen = scene - mu
        var = jnp.mean(cen * cen, axis=-1, keepdims=True)
        h = cen * jax.lax.rsqrt(var + 1e-5) * g_ref[...] + lb_ref[...]
        hb = h.astype(bf)
        gks = []
        for k in range(_H_CHUNKS):
            ffk = jnp.dot(hb, w1[:, k * ck:(k + 1) * ck],
                          preferred_element_type=jnp.float32)
            ffk = ffk.astype(bf) + b1[:, k * ck:(k + 1) * ck]
            half = jnp.asarray(0.5, bf) * ffk
            gks.append(half + half * jax.lax.erf(
                ffk * jnp.asarray(0.7071067811865476, bf)))
        ff = jnp.concatenate(gks, axis=1)
        acc = jnp.dot(ff, w2, preferred_element_type=jnp.float32)
        out_ref[sl, :] = h + acc + b2_ref[...]


@functools.partial(jax.jit, static_argnames=("block_n",))
def _i2st(x, mask, scene_embed, W_proj, b_proj, ln_g, ln_b, W1, b1, W2, b2,
          block_n=4096):
    n, c = x.shape
    h_dim = W1.shape[1]
    grid = (n // block_n,)
    row_spec = pl.BlockSpec((block_n, c), lambda i: (i, 0))
    full = lambda a: pl.BlockSpec(a.shape, lambda i: (0,) * a.ndim)
    return pl.pallas_call(
        _i2st_block,
        grid=grid,
        in_specs=[
            row_spec,
            pl.BlockSpec((block_n, 1), lambda i: (i, 0)),
            row_spec,
            full(W_proj), full(b_proj), full(ln_g), full(ln_b),
            full(W1), full(b1), full(W2), full(b2),
        ],
        out_specs=row_spec,
        out_shape=jax.ShapeDtypeStruct((n, c), jnp.float32),
    )(x, mask, scene_embed, W_proj, b_proj, ln_g, ln_b, W1, b1, W2, b2)


def kernel(x, fov_mask, scene_embed, W_proj, b_proj, ln_g, ln_b, W1, b1, W2, b2):
    b, n, c = x.shape
    h_dim = W1.shape[1]
    x2 = x.reshape(b * n, c).astype(jnp.bfloat16)
    mask = fov_mask.reshape(b * n, 1).astype(jnp.float32)
    se = jnp.broadcast_to(scene_embed[None], (b, n, c)).reshape(b * n, c)
    bf = jnp.bfloat16
    out = _i2st(x2, mask, se,
                W_proj.astype(bf), b_proj.reshape(1, c), ln_g.reshape(1, c),
                ln_b.reshape(1, c), W1.astype(bf), b1.reshape(1, h_dim).astype(bf),
                W2.astype(bf), b2.reshape(1, c))
    return out.reshape(b, n, c)


# speedup vs baseline: 1.7426x; 1.1707x over previous
import functools

import jax
import jax.numpy as jnp
from jax.experimental import pallas as pl
from jax.experimental.pallas import tpu as pltpu


_ROW_SPLIT = 1
_H_CHUNKS = 4


def _i2st_block(x_ref, m_ref, se_ref, wp_ref, bp_ref, g_ref, lb_ref,
                w1_ref, b1_ref, w2_ref, b2_ref, out_ref):
    bf = jnp.bfloat16
    wp = wp_ref[...]
    w1 = w1_ref[...]
    w2 = w2_ref[...]
    b1 = b1_ref[...]
    a = jnp.asarray(0.7978845608028654, bf)
    b = jnp.asarray(0.7978845608028654 * 0.044715, bf)
    rows = x_ref.shape[0] // _ROW_SPLIT
    ck = w1.shape[1] // _H_CHUNKS
    for r in range(_ROW_SPLIT):
        sl = pl.ds(r * rows, rows)
        proj = jnp.dot(x_ref[sl, :].astype(bf), wp,
                       preferred_element_type=jnp.float32)
        proj = proj + bp_ref[...]
        scene = jnp.where(m_ref[sl, :], proj, se_ref[sl, :])
        mu = jnp.mean(scene, axis=-1, keepdims=True)
        cen = scene - mu
        var = jnp.mean(cen * cen, axis=-1, keepdims=True)
        h = cen * jax.lax.rsqrt(var + 1e-5) * g_ref[...] + lb_ref[...]
        hb = h.astype(bf)
        gks = []
        for k in range(_H_CHUNKS):
            ffk = jnp.dot(hb, w1[:, k * ck:(k + 1) * ck],
                          preferred_element_type=jnp.float32)
            ffk = ffk.astype(bf) + b1[:, k * ck:(k + 1) * ck]
            half = jnp.asarray(0.5, bf) * ffk
            gks.append(half + half * jax.lax.erf(
                ffk * jnp.asarray(0.7071067811865476, bf)))
        ff = jnp.concatenate(gks, axis=1)
        acc = jnp.dot(ff, w2, preferred_element_type=jnp.float32)
        out_ref[sl, :] = h + acc + b2_ref[...]


@functools.partial(jax.jit, static_argnames=("block_n",))
def _i2st(x, mask, scene_embed, W_proj, b_proj, ln_g, ln_b, W1, b1, W2, b2,
          block_n=4096):
    n, c = x.shape
    h_dim = W1.shape[1]
    grid = (n // block_n,)
    row_spec = pl.BlockSpec((block_n, c), lambda i: (i, 0))
    full = lambda a: pl.BlockSpec(a.shape, lambda i: (0,) * a.ndim)
    return pl.pallas_call(
        _i2st_block,
        grid=grid,
        in_specs=[
            row_spec,
            pl.BlockSpec((block_n, 1), lambda i: (i, 0)),
            row_spec,
            full(W_proj), full(b_proj), full(ln_g), full(ln_b),
            full(W1), full(b1), full(W2), full(b2),
        ],
        out_specs=row_spec,
        out_shape=jax.ShapeDtypeStruct((n, c), jnp.float32),
    )(x, mask, scene_embed, W_proj, b_proj, ln_g, ln_b, W1, b1, W2, b2)


def kernel(x, fov_mask, scene_embed, W_proj, b_proj, ln_g, ln_b, W1, b1, W2, b2):
    b, n, c = x.shape
    h_dim = W1.shape[1]
    x2 = x.reshape(b * n, c)
    mask = fov_mask.reshape(b * n, 1)
    se = jnp.broadcast_to(scene_embed[None], (b, n, c)).reshape(b * n, c)
    bf = jnp.bfloat16
    out = _i2st(x2, mask, se,
                W_proj.astype(bf), b_proj.reshape(1, c), ln_g.reshape(1, c),
                ln_b.reshape(1, c), W1.astype(bf), b1.reshape(1, h_dim).astype(bf),
                W2.astype(bf), b2.reshape(1, c))
    return out.reshape(b, n, c)
